# Initial kernel scaffold; baseline (speedup 1.0000x reference)
import jax
import jax.numpy as jnp
from jax.experimental import pallas as pl


def kernel(local_xs, local_x_domain, local_x_cate, local_x_chn, local_x_seg_title, local_x_pid, local_x_semantic_emb, nonlocal_xs, nonlocal_x_domain, nonlocal_x_cate, nonlocal_x_chn, nonlocal_x_seg_title, nonlocal_x_pid, nonlocal_x_semantic_emb, ys, y_domain, y_cate, y_chn, y_seg_title, y_pid, y_semantic_emb, os, hashed_uid, zip_code, W_domain, W_cate, W_chn, W_seg_title, W_os, W_doc, W_uid, W_zip, W_pid):
    raise NotImplementedError("write your pallas kernel here")


# speedup vs baseline: 11.5430x; 11.5430x over previous
import functools

import jax
import jax.numpy as jnp
from jax import lax
from jax.experimental import pallas as pl
from jax.experimental.pallas import tpu as pltpu
from jax.experimental.pallas import tpu_sc as plsc

B = 4096
H = 50
N = B * H
NC, NS = 2, 16
NW = NC * NS
CX = 128
RPW = N // NW
NCHUNK = RPW // CX
BPW = B // NW

_f32 = jnp.float32
_i32 = jnp.int32


def _make_kernel():
    mesh = plsc.VectorSubcoreMesh(
        core_axis_name="c", subcore_axis_name="s", num_cores=NC, num_subcores=NS
    )
    out_type = (
        jax.ShapeDtypeStruct((N, 144), _f32),
        jax.ShapeDtypeStruct((N, 144), _f32),
        jax.ShapeDtypeStruct((B, 144), _f32),
        jax.ShapeDtypeStruct((B, 8), _f32),
        jax.ShapeDtypeStruct((B, 32), _f32),
        jax.ShapeDtypeStruct((B, 16), _f32),
    )
    scratch = [
        pltpu.VMEM((CX,), _i32),
        pltpu.VMEM((CX,), _i32),
        pltpu.VMEM((CX,), _i32),
        pltpu.VMEM((5 * CX,), _i32),
        pltpu.VMEM((5 * CX,), _i32),
        pltpu.VMEM((10 * CX,), _i32),
        pltpu.VMEM((5 * CX,), _i32),
        pltpu.VMEM((CX, 32), _f32),
        pltpu.VMEM((CX, 16), _f32),
        pltpu.VMEM((5 * CX, 16), _f32),
        pltpu.VMEM((5 * CX, 16), _f32),
        pltpu.VMEM((10 * CX, 16), _f32),
        pltpu.VMEM((5 * CX, 16), _f32),
        pltpu.VMEM((CX, 32), _f32),
        pltpu.VMEM((CX, 144), _f32),
        pltpu.VMEM((CX, 8), _f32),
        pltpu.VMEM((CX, 32), _f32),
        pltpu.VMEM((CX, 16), _f32),
        pltpu.SemaphoreType.DMA,
    ]

    @functools.partial(
        pl.kernel, mesh=mesh, out_type=out_type, scratch_types=scratch,
        name="ps_embed_sc",
        compiler_params=pltpu.CompilerParams(use_tc_tiling_on_sc=False),
    )
    def k(
        l_xs, l_dom, l_cate, l_chn, l_seg, l_pid, l_sem,
        n_xs, n_dom, n_cate, n_chn, n_seg, n_pid, n_sem,
        ys, y_dom, y_cate, y_chn, y_seg, y_pid, y_sem,
        os_i, uid_i, zip_i,
        W_domain, W_cate, W_chn, W_seg, W_os, W_doc, W_uid, W_zip, W_pid,
        o_lx, o_nx, o_y, o_os, o_uid, o_zip,
        i_a, i_b, i_c, i_cate, i_chn, i_seg, i_pid,
        g_doc, g_dom, g_cate, g_chn, g_seg, g_pid, g_sem,
        obuf, g_os, g_uid, g_zip, dsem,
    ):
        wid = lax.axis_index("s") * NC + lax.axis_index("c")

        def do_chunk(base, xs_r, dom_r, cate_r, chn_r, seg_r, pid_r,
                     sem_r, out_r, pid_scale):
            b5 = base * 5
            b10 = base * 10
            pltpu.sync_copy(xs_r.at[pl.ds(base, CX)], i_a)
            pltpu.sync_copy(dom_r.at[pl.ds(base, CX)], i_b)
            pltpu.sync_copy(cate_r.at[pl.ds(b5, 5 * CX)], i_cate)
            pltpu.sync_copy(chn_r.at[pl.ds(b5, 5 * CX)], i_chn)
            pltpu.sync_copy(seg_r.at[pl.ds(b10, 10 * CX)], i_seg)
            pltpu.sync_copy(pid_r.at[pl.ds(b5, 5 * CX)], i_pid)
            ds = []
            ds.append(pltpu.async_copy(W_doc.at[i_a], g_doc, dsem))
            ds.append(pltpu.async_copy(W_domain.at[i_b], g_dom, dsem))
            for j in range(5):
                ds.append(pltpu.async_copy(
                    W_cate.at[i_cate.at[pl.ds(j * CX, CX)]],
                    g_cate.at[pl.ds(j * CX, CX)], dsem))
            for j in range(5):
                ds.append(pltpu.async_copy(
                    W_chn.at[i_chn.at[pl.ds(j * CX, CX)]],
                    g_chn.at[pl.ds(j * CX, CX)], dsem))
            for j in range(10):
                ds.append(pltpu.async_copy(
                    W_seg.at[i_seg.at[pl.ds(j * CX, CX)]],
                    g_seg.at[pl.ds(j * CX, CX)], dsem))
            for j in range(5):
                ds.append(pltpu.async_copy(
                    W_pid.at[i_pid.at[pl.ds(j * CX, CX)]],
                    g_pid.at[pl.ds(j * CX, CX)], dsem))
            pltpu.sync_copy(sem_r.at[pl.ds(base, CX)], g_sem)
            for d in ds:
                d.wait()

            @pl.loop(0, CX)
            def _(i):
                obuf[i, pl.ds(0, 16)] = g_doc[i, pl.ds(0, 16)]
                obuf[i, pl.ds(16, 16)] = g_doc[i, pl.ds(16, 16)]
                obuf[i, pl.ds(32, 16)] = g_dom[i]
                i5 = 5 * i
                i10 = 10 * i
                c = ((g_cate[i5] + g_cate[i5 + 1])
                     + (g_cate[i5 + 2] + g_cate[i5 + 3])
                     + g_cate[i5 + 4])
                obuf[i, pl.ds(48, 16)] = c
                h = ((g_chn[i5] + g_chn[i5 + 1])
                     + (g_chn[i5 + 2] + g_chn[i5 + 3])
                     + g_chn[i5 + 4])
                obuf[i, pl.ds(64, 16)] = h
                s = (((g_seg[i10] + g_seg[i10 + 1])
                      + (g_seg[i10 + 2] + g_seg[i10 + 3]))
                     + ((g_seg[i10 + 4] + g_seg[i10 + 5])
                        + (g_seg[i10 + 6] + g_seg[i10 + 7]))
                     + (g_seg[i10 + 8] + g_seg[i10 + 9]))
                obuf[i, pl.ds(80, 16)] = s
                obuf[i, pl.ds(96, 16)] = g_sem[i, pl.ds(0, 16)]
                obuf[i, pl.ds(112, 16)] = g_sem[i, pl.ds(16, 16)]
                p = ((g_pid[i5] + g_pid[i5 + 1])
                     + (g_pid[i5 + 2] + g_pid[i5 + 3])
                     + g_pid[i5 + 4])
                obuf[i, pl.ds(128, 16)] = p * pid_scale
            pltpu.sync_copy(obuf, out_r.at[pl.ds(base, CX)])

        for (xs_r, dom_r, cate_r, chn_r, seg_r, pid_r, sem_r, out_r) in (
            (l_xs, l_dom, l_cate, l_chn, l_seg, l_pid, l_sem, o_lx),
            (n_xs, n_dom, n_cate, n_chn, n_seg, n_pid, n_sem, o_nx),
        ):
            @pl.loop(0, NCHUNK)
            def _(cidx, xs_r=xs_r, dom_r=dom_r, cate_r=cate_r, chn_r=chn_r,
                  seg_r=seg_r, pid_r=pid_r, sem_r=sem_r, out_r=out_r):
                do_chunk(wid * RPW + cidx * CX, xs_r, dom_r, cate_r, chn_r,
                         seg_r, pid_r, sem_r, out_r, _f32(0.2))

        do_chunk(wid * BPW, ys, y_dom, y_cate, y_chn, y_seg, y_pid,
                 y_sem, o_y, _f32(1.0))

        base = wid * BPW
        pltpu.sync_copy(os_i.at[pl.ds(base, CX)], i_a)
        pltpu.sync_copy(uid_i.at[pl.ds(base, CX)], i_b)
        pltpu.sync_copy(zip_i.at[pl.ds(base, CX)], i_c)
        @pl.loop(0, CX // 16)
        def _(kk):
            v = i_c[pl.ds(kk * 16, 16)]
            i_c[pl.ds(kk * 16, 16)] = jnp.minimum(
                jnp.maximum(v, 0), jnp.int32(99999))
        d1 = pltpu.async_copy(W_os.at[i_a], g_os, dsem)
        d2 = pltpu.async_copy(W_uid.at[i_b], g_uid, dsem)
        d3 = pltpu.async_copy(W_zip.at[i_c], g_zip, dsem)
        d1.wait(); d2.wait(); d3.wait()
        pltpu.sync_copy(g_os, o_os.at[pl.ds(base, CX)])
        pltpu.sync_copy(g_uid, o_uid.at[pl.ds(base, CX)])
        pltpu.sync_copy(g_zip, o_zip.at[pl.ds(base, CX)])

    return k


_KERNEL = None


def _get_kernel():
    global _KERNEL
    if _KERNEL is None:
        _KERNEL = _make_kernel()
    return _KERNEL


def kernel(local_xs, local_x_domain, local_x_cate, local_x_chn,
           local_x_seg_title, local_x_pid, local_x_semantic_emb,
           nonlocal_xs, nonlocal_x_domain, nonlocal_x_cate, nonlocal_x_chn,
           nonlocal_x_seg_title, nonlocal_x_pid, nonlocal_x_semantic_emb,
           ys, y_domain, y_cate, y_chn, y_seg_title, y_pid, y_semantic_emb,
           os, hashed_uid, zip_code,
           W_domain, W_cate, W_chn, W_seg_title, W_os, W_doc, W_uid, W_zip,
           W_pid):
    k = _get_kernel()
    flat = lambda a: a.reshape(-1)
    lx, nx, y, os_e, uid_e, zip_e = k(
        flat(local_xs), flat(local_x_domain), flat(local_x_cate),
        flat(local_x_chn), flat(local_x_seg_title), flat(local_x_pid),
        local_x_semantic_emb.reshape(N, 32),
        flat(nonlocal_xs), flat(nonlocal_x_domain), flat(nonlocal_x_cate),
        flat(nonlocal_x_chn), flat(nonlocal_x_seg_title), flat(nonlocal_x_pid),
        nonlocal_x_semantic_emb.reshape(N, 32),
        flat(ys), flat(y_domain), flat(y_cate), flat(y_chn),
        flat(y_seg_title), flat(y_pid), y_semantic_emb,
        flat(os), flat(hashed_uid), flat(zip_code),
        W_domain, W_cate, W_chn, W_seg_title, W_os, W_doc, W_uid, W_zip,
        W_pid,
    )
    return (lx.reshape(B, H, 144), nx.reshape(B, H, 144), y,
            os_e, uid_e, zip_e)


# speedup vs baseline: 15.4134x; 1.3353x over previous
import functools

import jax
import jax.numpy as jnp
from jax import lax
from jax.experimental import pallas as pl
from jax.experimental.pallas import tpu as pltpu
from jax.experimental.pallas import tpu_sc as plsc

B = 4096
H = 50
N = B * H
NC, NS = 2, 16
NW = NC * NS
CX = 64
RPW = N // NW
NCHUNK = RPW // CX
BPW = B // NW

_f32 = jnp.float32
_i32 = jnp.int32


def _make_kernel():
    mesh = plsc.VectorSubcoreMesh(
        core_axis_name="c", subcore_axis_name="s", num_cores=NC, num_subcores=NS
    )
    out_type = (
        jax.ShapeDtypeStruct((N * 144,), _f32),
        jax.ShapeDtypeStruct((N * 144,), _f32),
        jax.ShapeDtypeStruct((B * 144,), _f32),
        jax.ShapeDtypeStruct((B, 8), _f32),
        jax.ShapeDtypeStruct((B, 32), _f32),
        jax.ShapeDtypeStruct((B, 16), _f32),
    )
    idx_set = [
        pltpu.VMEM((CX,), _i32),
        pltpu.VMEM((CX,), _i32),
        pltpu.VMEM((5 * CX,), _i32),
        pltpu.VMEM((5 * CX,), _i32),
        pltpu.VMEM((10 * CX,), _i32),
        pltpu.VMEM((5 * CX,), _i32),
    ]
    g_set = [
        pltpu.VMEM((CX, 32), _f32),
        pltpu.VMEM((CX, 16), _f32),
        pltpu.VMEM((5 * CX, 16), _f32),
        pltpu.VMEM((5 * CX, 16), _f32),
        pltpu.VMEM((10 * CX, 16), _f32),
        pltpu.VMEM((5 * CX, 16), _f32),
        pltpu.VMEM((CX * 32,), _f32),
    ]
    scratch = (
        idx_set + idx_set + g_set + g_set
        + [pltpu.VMEM((CX * 144,), _f32)] * 2
        + [pltpu.VMEM((CX, 8), _f32),
           pltpu.VMEM((CX, 32), _f32),
           pltpu.VMEM((CX, 16), _f32)]
        + [pltpu.SemaphoreType.DMA] * 6
    )

    @functools.partial(
        pl.kernel, mesh=mesh, out_type=out_type, scratch_types=scratch,
        name="ps_embed_sc",
        compiler_params=pltpu.CompilerParams(use_tc_tiling_on_sc=False),
    )
    def k(
        l_xs, l_dom, l_cate, l_chn, l_seg, l_pid, l_sem,
        n_xs, n_dom, n_cate, n_chn, n_seg, n_pid, n_sem,
        ys, y_dom, y_cate, y_chn, y_seg, y_pid, y_sem,
        os_i, uid_i, zip_i,
        W_domain, W_cate, W_chn, W_seg, W_os, W_doc, W_uid, W_zip, W_pid,
        o_lx, o_nx, o_y, o_os, o_uid, o_zip,
        ia0, ib0, ic0, ih0, is0, ip0,
        ia1, ib1, ic1, ih1, is1, ip1,
        gd0, gm0, gc0, gh0, gs0, gp0, ge0,
        gd1, gm1, gc1, gh1, gs1, gp1, ge1,
        ob0, ob1,
        s_os, s_uid, s_zip,
        isem0, isem1, gsem0, gsem1, osem0, osem1,
    ):
        wid = lax.axis_index("s") * NC + lax.axis_index("c")
        IDX = ((ia0, ib0, ic0, ih0, is0, ip0), (ia1, ib1, ic1, ih1, is1, ip1))
        G = ((gd0, gm0, gc0, gh0, gs0, gp0, ge0),
             (gd1, gm1, gc1, gh1, gs1, gp1, ge1))
        OB = (ob0, ob1)
        ISEM = (isem0, isem1)
        GSEM = (gsem0, gsem1)
        OSEM = (osem0, osem1)

        def idx_copies(base, p, feats):
            xs_r, dom_r, cate_r, chn_r, seg_r, pid_r = feats
            ia, ib, ic, ih, isg, ip = IDX[p]
            return [
                pltpu.make_async_copy(xs_r.at[pl.ds(base, CX)], ia, ISEM[p]),
                pltpu.make_async_copy(dom_r.at[pl.ds(base, CX)], ib, ISEM[p]),
                pltpu.make_async_copy(cate_r.at[pl.ds(5 * base, 5 * CX)], ic,
                                      ISEM[p]),
                pltpu.make_async_copy(chn_r.at[pl.ds(5 * base, 5 * CX)], ih,
                                      ISEM[p]),
                pltpu.make_async_copy(seg_r.at[pl.ds(10 * base, 10 * CX)], isg,
                                      ISEM[p]),
                pltpu.make_async_copy(pid_r.at[pl.ds(5 * base, 5 * CX)], ip,
                                      ISEM[p]),
            ]

        def gather_copies(base, p, sem_r, tables):
            W_dom_r, W_cate_r, W_chn_r, W_seg_r, W_doc_r, W_pid_r = tables
            ia, ib, ic, ih, isg, ip = IDX[p]
            gd, gm, gc, gh, gs, gp, ge = G[p]
            sem = GSEM[p]
            cps = [
                pltpu.make_async_copy(W_doc_r.at[ia], gd, sem),
                pltpu.make_async_copy(W_dom_r.at[ib], gm, sem),
            ]
            for j in range(0, 5 * CX, 128):
                nrun = min(128, 5 * CX - j)
                cps.append(pltpu.make_async_copy(
                    W_cate_r.at[ic.at[pl.ds(j, nrun)]],
                    gc.at[pl.ds(j, nrun)], sem))
                cps.append(pltpu.make_async_copy(
                    W_chn_r.at[ih.at[pl.ds(j, nrun)]],
                    gh.at[pl.ds(j, nrun)], sem))
                cps.append(pltpu.make_async_copy(
                    W_pid_r.at[ip.at[pl.ds(j, nrun)]],
                    gp.at[pl.ds(j, nrun)], sem))
            for j in range(0, 10 * CX, 128):
                nrun = min(128, 10 * CX - j)
                cps.append(pltpu.make_async_copy(
                    W_seg_r.at[isg.at[pl.ds(j, nrun)]],
                    gs.at[pl.ds(j, nrun)], sem))
            cps.append(pltpu.make_async_copy(
                sem_r.at[pl.ds(32 * base, 32 * CX)], ge, sem))
            return cps

        def compute(p, q, pid_scale):
            gd, gm, gc, gh, gs, gp, ge = G[p]
            ob = OB[q]

            @pl.loop(0, CX)
            def _(i):
                off = 144 * i
                i5 = 5 * i
                i10 = 10 * i
                ob[pl.ds(off, 16)] = gd[i, pl.ds(0, 16)]
                ob[pl.ds(off + 16, 16)] = gd[i, pl.ds(16, 16)]
                ob[pl.ds(off + 32, 16)] = gm[i]
                c = ((gc[i5] + gc[i5 + 1]) + (gc[i5 + 2] + gc[i5 + 3])
                     + gc[i5 + 4])
                ob[pl.ds(off + 48, 16)] = c
                h = ((gh[i5] + gh[i5 + 1]) + (gh[i5 + 2] + gh[i5 + 3])
                     + gh[i5 + 4])
                ob[pl.ds(off + 64, 16)] = h
                s = (((gs[i10] + gs[i10 + 1]) + (gs[i10 + 2] + gs[i10 + 3]))
                     + ((gs[i10 + 4] + gs[i10 + 5])
                        + (gs[i10 + 6] + gs[i10 + 7]))
                     + (gs[i10 + 8] + gs[i10 + 9]))
                ob[pl.ds(off + 80, 16)] = s
                ob[pl.ds(off + 96, 16)] = ge[pl.ds(32 * i, 16)]
                ob[pl.ds(off + 112, 16)] = ge[pl.ds(32 * i + 16, 16)]
                pp = ((gp[i5] + gp[i5 + 1]) + (gp[i5 + 2] + gp[i5 + 3])
                      + gp[i5 + 4])
                ob[pl.ds(off + 128, 16)] = pp * pid_scale

        def out_copy(base, q, out_r):
            return pltpu.make_async_copy(
                OB[q], out_r.at[pl.ds(144 * base, 144 * CX)], OSEM[q])

        TABLES = (W_domain, W_cate, W_chn, W_seg, W_doc, W_pid)

        for (feats, sem_r, out_r) in (
            ((l_xs, l_dom, l_cate, l_chn, l_seg, l_pid), l_sem, o_lx),
            ((n_xs, n_dom, n_cate, n_chn, n_seg, n_pid), n_sem, o_nx),
        ):
            x0 = wid * RPW

            def pre(t, pj):
                for d in idx_copies(x0 + t * CX, pj, feats):
                    d.start()
                for d in idx_copies(x0 + t * CX, pj, feats):
                    d.wait()
                for d in gather_copies(x0 + t * CX, pj, sem_r, TABLES):
                    d.start()

            def run(t, pj, with_wait_out, fire_next):
                for d in gather_copies(x0 + t * CX, pj, sem_r, TABLES):
                    d.wait()
                if fire_next:
                    for d in idx_copies(x0 + (t + 2) * CX, pj, feats):
                        d.start()
                if with_wait_out:
                    out_copy(x0 + (t - 2) * CX, pj, out_r).wait()
                compute(pj, pj, _f32(0.2))
                out_copy(x0 + t * CX, pj, out_r).start()
                if fire_next:
                    for d in idx_copies(x0 + (t + 2) * CX, pj, feats):
                        d.wait()
                    for d in gather_copies(x0 + (t + 2) * CX, pj, sem_r,
                                           TABLES):
                        d.start()

            pre(0, 0)
            pre(1, 1)
            run(0, 0, False, True)
            run(1, 1, False, True)

            @pl.loop(2, NCHUNK - 2, step=2)
            def _(t):
                run(t, 0, True, True)
                run(t + 1, 1, True, True)

            run(NCHUNK - 2, 0, True, False)
            run(NCHUNK - 1, 1, True, False)
            out_copy(x0 + (NCHUNK - 2) * CX, 0, out_r).wait()
            out_copy(x0 + (NCHUNK - 1) * CX, 1, out_r).wait()

        yfeats = (ys, y_dom, y_cate, y_chn, y_seg, y_pid)
        for t in range(2):
            base = wid * BPW + t * CX
            p = t & 1
            for d in idx_copies(base, p, yfeats):
                d.start()
            for d in idx_copies(base, p, yfeats):
                d.wait()
            for d in gather_copies(base, p, y_sem, TABLES):
                d.start()
            for d in gather_copies(base, p, y_sem, TABLES):
                d.wait()
            compute(p, p, _f32(1.0))
            out_copy(base, p, o_y).start()
            out_copy(base, p, o_y).wait()

        for t in range(2):
            base = wid * BPW + t * CX
            ia, ib, ic = IDX[0][0], IDX[0][1], IDX[1][0]
            pltpu.sync_copy(os_i.at[pl.ds(base, CX)], ia)
            pltpu.sync_copy(uid_i.at[pl.ds(base, CX)], ib)
            pltpu.sync_copy(zip_i.at[pl.ds(base, CX)], ic)

            @pl.loop(0, CX // 16)
            def _(kk):
                v = ic[pl.ds(kk * 16, 16)]
                ic[pl.ds(kk * 16, 16)] = jnp.minimum(
                    jnp.maximum(v, 0), jnp.int32(99999))

            d1 = pltpu.async_copy(W_os.at[ia], s_os, gsem0)
            d2 = pltpu.async_copy(W_uid.at[ib], s_uid, gsem0)
            d3 = pltpu.async_copy(W_zip.at[ic], s_zip, gsem0)
            d1.wait(); d2.wait(); d3.wait()
            pltpu.sync_copy(s_os, o_os.at[pl.ds(base, CX)])
            pltpu.sync_copy(s_uid, o_uid.at[pl.ds(base, CX)])
            pltpu.sync_copy(s_zip, o_zip.at[pl.ds(base, CX)])

    return k


_KERNEL = None


def _get_kernel():
    global _KERNEL
    if _KERNEL is None:
        _KERNEL = _make_kernel()
    return _KERNEL


def kernel(local_xs, local_x_domain, local_x_cate, local_x_chn,
           local_x_seg_title, local_x_pid, local_x_semantic_emb,
           nonlocal_xs, nonlocal_x_domain, nonlocal_x_cate, nonlocal_x_chn,
           nonlocal_x_seg_title, nonlocal_x_pid, nonlocal_x_semantic_emb,
           ys, y_domain, y_cate, y_chn, y_seg_title, y_pid, y_semantic_emb,
           os, hashed_uid, zip_code,
           W_domain, W_cate, W_chn, W_seg_title, W_os, W_doc, W_uid, W_zip,
           W_pid):
    k = _get_kernel()
    flat = lambda a: a.reshape(-1)
    lx, nx, y, os_e, uid_e, zip_e = k(
        flat(local_xs), flat(local_x_domain), flat(local_x_cate),
        flat(local_x_chn), flat(local_x_seg_title), flat(local_x_pid),
        flat(local_x_semantic_emb),
        flat(nonlocal_xs), flat(nonlocal_x_domain), flat(nonlocal_x_cate),
        flat(nonlocal_x_chn), flat(nonlocal_x_seg_title), flat(nonlocal_x_pid),
        flat(nonlocal_x_semantic_emb),
        flat(ys), flat(y_domain), flat(y_cate), flat(y_chn),
        flat(y_seg_title), flat(y_pid), flat(y_semantic_emb),
        flat(os), flat(hashed_uid), flat(zip_code),
        W_domain, W_cate, W_chn, W_seg_title, W_os, W_doc, W_uid, W_zip,
        W_pid,
    )
    return (lx.reshape(B, H, 144), nx.reshape(B, H, 144), y.reshape(B, 144),
            os_e, uid_e, zip_e)
